# baseline (speedup 1.0000x reference)
import functools

import jax
import jax.numpy as jnp
from jax import lax
from jax.experimental import pallas as pl
from jax.experimental.pallas import tpu as pltpu
from jax.experimental.pallas import tpu_sc as plsc

N = 10000
E = 320000
D_IN = 128
F1 = 16
C = 7

NC = 2
NS = 16
NW = NC * NS
EPW = E // NW
CH = 400
G = CH // 16
NCHUNK = EPW // CH
NPAD = 10240
RPT = NPAD // NS

_f32 = jnp.float32


def _sc_layer(roww, with_den):
    mesh = plsc.VectorSubcoreMesh(
        core_axis_name="c", subcore_axis_name="s", num_cores=NC, num_subcores=NS
    )

    nbuf = 4

    out_type = [jax.ShapeDtypeStruct((NC, NPAD, roww), _f32)]
    scratch = [
        pltpu.VMEM((N,), _f32),
        pltpu.VMEM((N,), _f32),
        pltpu.VMEM((NCHUNK, CH), jnp.int32),
        pltpu.VMEM((NCHUNK, CH), jnp.int32),
        pltpu.VMEM((nbuf, CH, roww), _f32),
        pltpu.VMEM((nbuf, CH, roww), _f32),
        pltpu.VMEM_SHARED((NPAD, roww), _f32),
        [pltpu.SemaphoreType.DMA] * nbuf,
        [pltpu.SemaphoreType.DMA] * nbuf,
        [pltpu.SemaphoreType.DMA] * 6,
    ]
    if with_den:
        out_type.append(jax.ShapeDtypeStruct((NC, NS, N), _f32))
        scratch.insert(7, pltpu.VMEM((N,), _f32))
    else:
        out_type = out_type[0]

    @functools.partial(
        pl.kernel,
        out_type=out_type,
        mesh=mesh,
        scratch_types=scratch,
        compiler_params=pltpu.CompilerParams(
            use_tc_tiling_on_sc=False, needs_layout_passes=False
        ),
    )
    def layer(t_hbm, es_hbm, ed_hbm, src_hbm, dst_hbm, z_hbm, *rest):
        if with_den:
            (zden_hbm, out_hbm, den_hbm,
             es_v, ed_v, src_v, dst_v, gbuf, sbuf, acc_sh, den_v,
             gsem, ssem, psem) = rest
        else:
            (out_hbm,
             es_v, ed_v, src_v, dst_v, gbuf, sbuf, acc_sh,
             gsem, ssem, psem) = rest
        cid = lax.axis_index("c")
        sid = lax.axis_index("s")
        wid = cid * NS + sid

        setups = [
            (es_hbm, es_v),
            (ed_hbm, ed_v),
            (src_hbm.at[wid], src_v),
            (dst_hbm.at[wid], dst_v),
            (z_hbm, acc_sh.at[pl.ds(sid * RPT, RPT)]),
        ]
        if with_den:
            setups.append((zden_hbm, den_v))
        for k, (s, d) in enumerate(setups):
            pltpu.async_copy(s, d, psem[k])
        for k, (s, d) in enumerate(setups):
            pltpu.make_async_copy(s, d, psem[k]).wait()
        plsc.subcore_barrier()

        def gather(c, b):
            pltpu.async_copy(t_hbm.at[src_v.at[c]], gbuf.at[b], gsem[b])

        offs = list(range(0, roww - 15, 16))
        if roww % 16:
            offs.append(roww - 16)

        def compute(c, b):
            src_row = src_v.at[c]
            dst_row = dst_v.at[c]
            for g in range(G):
                s16 = src_row[pl.ds(g * 16, 16)]
                d16 = dst_row[pl.ds(g * 16, 16)]
                x = plsc.load_gather(es_v, [s16]) + plsc.load_gather(ed_v, [d16])
                ex16 = jnp.exp(jnp.maximum(x, 0.2 * x))
                if with_den:
                    plsc.addupdate_scatter(den_v, [d16], ex16)
                for i in range(16):
                    e = g * 16 + i
                    exi = ex16[i]
                    for h in offs:
                        sbuf[b, e, pl.ds(h, 16)] = (
                            gbuf[b, e, pl.ds(h, 16)] * exi
                        )

        def scatter(c, b):
            pltpu.async_copy(sbuf.at[b], acc_sh.at[dst_v.at[c]], ssem[b], add=True)

        for b in range(nbuf):
            gather(b, b)

        nfull = (NCHUNK - 1) // nbuf

        def step(t, carry):
            for b in range(nbuf):
                c = t * nbuf + b
                pltpu.make_async_copy(t_hbm.at[src_v.at[0]], gbuf.at[b],
                                      gsem[b]).wait()

                @pl.when(t > 0)
                def _():
                    pltpu.make_async_copy(
                        sbuf.at[b], acc_sh.at[dst_v.at[0]], ssem[b]).wait()

                compute(c, b)
                scatter(c, b)

                @pl.when(c + nbuf <= NCHUNK - 1)
                def _():
                    gather(c + nbuf, b)
            return carry

        lax.fori_loop(0, nfull, step, 0)

        last = NCHUNK - 1
        pltpu.make_async_copy(t_hbm.at[src_v.at[0]], gbuf.at[0], gsem[0]).wait()
        pltpu.make_async_copy(sbuf.at[0], acc_sh.at[dst_v.at[0]], ssem[0]).wait()
        compute(last, 0)
        scatter(last, 0)
        for b in range(nbuf):
            pltpu.make_async_copy(sbuf.at[b], acc_sh.at[dst_v.at[0]],
                                  ssem[b]).wait()

        plsc.subcore_barrier()
        outs = [(acc_sh.at[pl.ds(sid * RPT, RPT)],
                 out_hbm.at[cid, pl.ds(sid * RPT, RPT)])]
        if with_den:
            outs.append((den_v, den_hbm.at[cid, sid]))
        for k, (s, d) in enumerate(outs):
            pltpu.async_copy(s, d, psem[k])
        for k, (s, d) in enumerate(outs):
            pltpu.make_async_copy(s, d, psem[k]).wait()

    return layer


_sc_layer1 = _sc_layer(F1, with_den=True)
_sc_layer16 = _sc_layer(16, with_den=False)


def _prep1(h_ref, w1_ref, as_ref, ad_ref, t1_ref, es_ref, ed_ref):
    hw = jnp.dot(h_ref[...], w1_ref[...], preferred_element_type=_f32)
    t1_ref[...] = hw
    es_ref[...] = jnp.dot(hw, as_ref[...], preferred_element_type=_f32)
    ed_ref[...] = jnp.dot(hw, ad_ref[...], preferred_element_type=_f32)


def _mid(p_ref, den_ref, w2_ref, as_ref, ad_ref, h1_ref, t2_ref, es_ref, ed_ref):
    p = p_ref[...]
    s = (p[0] + p[1])[:N]
    den = jnp.sum(den_ref[...], axis=(0, 1))
    h1 = s / (den[:, None] + 1e-16)
    h1_ref[...] = h1
    hd = jnp.where(h1 > 0, h1, jnp.exp(h1) - 1.0)
    hw2 = jnp.dot(hd, w2_ref[...], preferred_element_type=_f32)
    ones = jnp.ones((N, 1), _f32)
    zeros = jnp.zeros((N, 16 - C - 1), _f32)
    t2_ref[...] = jnp.concatenate([hw2, ones, zeros], axis=1)
    es_ref[...] = jnp.dot(hw2, as_ref[...], preferred_element_type=_f32)
    ed_ref[...] = jnp.dot(hw2, ad_ref[...], preferred_element_type=_f32)


def _fin(p_ref, h2_ref, lp_ref):
    p = p_ref[...]
    s = (p[0] + p[1])[:N]
    h2 = s[:, :C] / (s[:, C:C + 1] + 1e-16)
    h2_ref[...] = h2
    m = jnp.max(h2, axis=1, keepdims=True)
    z = h2 - m
    lse = jnp.log(jnp.sum(jnp.exp(z), axis=1, keepdims=True))
    lp_ref[...] = z - lse


def kernel(H, edge_index, W1, a1_src, a1_dst, W2, a2_src, a2_dst):
    src = edge_index[0].reshape(NW, NCHUNK, CH)
    dst = edge_index[1].reshape(NW, NCHUNK, CH)

    t1, es1, ed1 = pl.pallas_call(
        _prep1,
        out_shape=(
            jax.ShapeDtypeStruct((N, F1), _f32),
            jax.ShapeDtypeStruct((N, 1), _f32),
            jax.ShapeDtypeStruct((N, 1), _f32),
        ),
    )(H, W1, a1_src.reshape(F1, 1), a1_dst.reshape(F1, 1))

    z16 = jnp.zeros((RPT, 16), _f32)
    zden = jnp.zeros((N,), _f32)
    part1, den1 = _sc_layer1(t1, es1.reshape(N), ed1.reshape(N), src, dst,
                             z16, zden)

    h1, t2, es2, ed2 = pl.pallas_call(
        _mid,
        out_shape=(
            jax.ShapeDtypeStruct((N, F1), _f32),
            jax.ShapeDtypeStruct((N, 16), _f32),
            jax.ShapeDtypeStruct((N, 1), _f32),
            jax.ShapeDtypeStruct((N, 1), _f32),
        ),
    )(part1, den1, W2, a2_src.reshape(C, 1), a2_dst.reshape(C, 1))

    part2 = _sc_layer16(t2, es2.reshape(N), ed2.reshape(N), src, dst, z16)

    h2, logp = pl.pallas_call(
        _fin,
        out_shape=(
            jax.ShapeDtypeStruct((N, C), _f32),
            jax.ShapeDtypeStruct((N, C), _f32),
        ),
    )(part2)

    return logp, (h1, h2)


# speedup vs baseline: 1.0541x; 1.0541x over previous
import functools

import jax
import jax.numpy as jnp
from jax import lax
from jax.experimental import pallas as pl
from jax.experimental.pallas import tpu as pltpu
from jax.experimental.pallas import tpu_sc as plsc

N = 10000
E = 320000
D_IN = 128
F1 = 16
C = 7

NC = 2
NS = 16
NW = NC * NS
EPW = E // NW
CH = 400
G = CH // 16
NCHUNK = EPW // CH
NPAD = 10240
RPT = NPAD // NS

_f32 = jnp.float32


def _sc_layer(roww, with_den):
    mesh = plsc.VectorSubcoreMesh(
        core_axis_name="c", subcore_axis_name="s", num_cores=NC, num_subcores=NS
    )

    nbuf = 2

    out_type = [jax.ShapeDtypeStruct((NC, NPAD, roww), _f32)]
    scratch = [
        pltpu.VMEM((N,), _f32),
        pltpu.VMEM((N,), _f32),
        pltpu.VMEM((NCHUNK, CH), jnp.int32),
        pltpu.VMEM((NCHUNK, CH), jnp.int32),
        pltpu.VMEM((nbuf, CH, roww), _f32),
        pltpu.VMEM((nbuf, CH, roww), _f32),
        pltpu.VMEM_SHARED((NPAD, roww), _f32),
        [pltpu.SemaphoreType.DMA] * nbuf,
        [pltpu.SemaphoreType.DMA] * nbuf,
        [pltpu.SemaphoreType.DMA] * 6,
    ]
    if with_den:
        out_type.append(jax.ShapeDtypeStruct((NC, NS, N), _f32))
        scratch.insert(7, pltpu.VMEM((N,), _f32))
    else:
        out_type = out_type[0]

    @functools.partial(
        pl.kernel,
        out_type=out_type,
        mesh=mesh,
        scratch_types=scratch,
        compiler_params=pltpu.CompilerParams(
            use_tc_tiling_on_sc=False, needs_layout_passes=False
        ),
    )
    def layer(t_hbm, es_hbm, ed_hbm, src_hbm, dst_hbm, z_hbm, *rest):
        if with_den:
            (zden_hbm, out_hbm, den_hbm,
             es_v, ed_v, src_v, dst_v, gbuf, sbuf, acc_sh, den_v,
             gsem, ssem, psem) = rest
        else:
            (out_hbm,
             es_v, ed_v, src_v, dst_v, gbuf, sbuf, acc_sh,
             gsem, ssem, psem) = rest
        cid = lax.axis_index("c")
        sid = lax.axis_index("s")
        wid = cid * NS + sid

        setups = [
            (es_hbm, es_v),
            (ed_hbm, ed_v),
            (src_hbm.at[wid], src_v),
            (dst_hbm.at[wid], dst_v),
            (z_hbm, acc_sh.at[pl.ds(sid * RPT, RPT)]),
        ]
        if with_den:
            setups.append((zden_hbm, den_v))
        for k, (s, d) in enumerate(setups):
            pltpu.async_copy(s, d, psem[k])
        for k, (s, d) in enumerate(setups):
            pltpu.make_async_copy(s, d, psem[k]).wait()
        plsc.subcore_barrier()

        def gather(c, b):
            pltpu.async_copy(t_hbm.at[src_v.at[c]], gbuf.at[b], gsem[b])

        offs = list(range(0, roww - 15, 16))
        if roww % 16:
            offs.append(roww - 16)

        def compute(c, b):
            src_row = src_v.at[c]
            dst_row = dst_v.at[c]
            for g in range(G):
                s16 = src_row[pl.ds(g * 16, 16)]
                d16 = dst_row[pl.ds(g * 16, 16)]
                x = plsc.load_gather(es_v, [s16]) + plsc.load_gather(ed_v, [d16])
                ex16 = jnp.exp(jnp.maximum(x, 0.2 * x))
                if with_den:
                    plsc.addupdate_scatter(den_v, [d16], ex16)
                for i in range(16):
                    e = g * 16 + i
                    exi = ex16[i]
                    for h in offs:
                        sbuf[b, e, pl.ds(h, 16)] = (
                            gbuf[b, e, pl.ds(h, 16)] * exi
                        )

        def scatter(c, b):
            pltpu.async_copy(sbuf.at[b], acc_sh.at[dst_v.at[c]], ssem[b], add=True)

        for b in range(nbuf):
            gather(b, b)

        nfull = (NCHUNK - 1) // nbuf

        def step(t, carry):
            for b in range(nbuf):
                c = t * nbuf + b
                pltpu.make_async_copy(t_hbm.at[src_v.at[0]], gbuf.at[b],
                                      gsem[b]).wait()

                @pl.when(t > 0)
                def _():
                    pltpu.make_async_copy(
                        sbuf.at[b], acc_sh.at[dst_v.at[0]], ssem[b]).wait()

                compute(c, b)
                scatter(c, b)

                @pl.when(c + nbuf <= NCHUNK - 1)
                def _():
                    gather(c + nbuf, b)
            return carry

        lax.fori_loop(0, nfull, step, 0)

        last = NCHUNK - 1
        pltpu.make_async_copy(t_hbm.at[src_v.at[0]], gbuf.at[0], gsem[0]).wait()
        pltpu.make_async_copy(sbuf.at[0], acc_sh.at[dst_v.at[0]], ssem[0]).wait()
        compute(last, 0)
        scatter(last, 0)
        for b in range(nbuf):
            pltpu.make_async_copy(sbuf.at[b], acc_sh.at[dst_v.at[0]],
                                  ssem[b]).wait()

        plsc.subcore_barrier()
        outs = [(acc_sh.at[pl.ds(sid * RPT, RPT)],
                 out_hbm.at[cid, pl.ds(sid * RPT, RPT)])]
        if with_den:
            outs.append((den_v, den_hbm.at[cid, sid]))
        for k, (s, d) in enumerate(outs):
            pltpu.async_copy(s, d, psem[k])
        for k, (s, d) in enumerate(outs):
            pltpu.make_async_copy(s, d, psem[k]).wait()

    return layer


_sc_layer1 = _sc_layer(F1, with_den=True)
_sc_layer16 = _sc_layer(16, with_den=False)


def _prep1(h_ref, w1_ref, as_ref, ad_ref, t1_ref, es_ref, ed_ref):
    hw = jnp.dot(h_ref[...], w1_ref[...], preferred_element_type=_f32)
    t1_ref[...] = hw
    es_ref[...] = jnp.dot(hw, as_ref[...], preferred_element_type=_f32)
    ed_ref[...] = jnp.dot(hw, ad_ref[...], preferred_element_type=_f32)


def _mid(p_ref, den_ref, w2_ref, as_ref, ad_ref, h1_ref, t2_ref, es_ref, ed_ref):
    p = p_ref[...]
    s = (p[0] + p[1])[:N]
    den = jnp.sum(den_ref[...], axis=(0, 1))
    h1 = s / (den[:, None] + 1e-16)
    h1_ref[...] = h1
    hd = jnp.where(h1 > 0, h1, jnp.exp(h1) - 1.0)
    hw2 = jnp.dot(hd, w2_ref[...], preferred_element_type=_f32)
    ones = jnp.ones((N, 1), _f32)
    zeros = jnp.zeros((N, 16 - C - 1), _f32)
    t2_ref[...] = jnp.concatenate([hw2, ones, zeros], axis=1)
    es_ref[...] = jnp.dot(hw2, as_ref[...], preferred_element_type=_f32)
    ed_ref[...] = jnp.dot(hw2, ad_ref[...], preferred_element_type=_f32)


def _fin(p_ref, h2_ref, lp_ref):
    p = p_ref[...]
    s = (p[0] + p[1])[:N]
    h2 = s[:, :C] / (s[:, C:C + 1] + 1e-16)
    h2_ref[...] = h2
    m = jnp.max(h2, axis=1, keepdims=True)
    z = h2 - m
    lse = jnp.log(jnp.sum(jnp.exp(z), axis=1, keepdims=True))
    lp_ref[...] = z - lse


def kernel(H, edge_index, W1, a1_src, a1_dst, W2, a2_src, a2_dst):
    src = edge_index[0].reshape(NW, NCHUNK, CH)
    dst = edge_index[1].reshape(NW, NCHUNK, CH)

    t1, es1, ed1 = pl.pallas_call(
        _prep1,
        out_shape=(
            jax.ShapeDtypeStruct((N, F1), _f32),
            jax.ShapeDtypeStruct((N, 1), _f32),
            jax.ShapeDtypeStruct((N, 1), _f32),
        ),
    )(H, W1, a1_src.reshape(F1, 1), a1_dst.reshape(F1, 1))

    z16 = jnp.zeros((RPT, 16), _f32)
    zden = jnp.zeros((N,), _f32)
    part1, den1 = _sc_layer1(t1, es1.reshape(N), ed1.reshape(N), src, dst,
                             z16, zden)

    h1, t2, es2, ed2 = pl.pallas_call(
        _mid,
        out_shape=(
            jax.ShapeDtypeStruct((N, F1), _f32),
            jax.ShapeDtypeStruct((N, 16), _f32),
            jax.ShapeDtypeStruct((N, 1), _f32),
            jax.ShapeDtypeStruct((N, 1), _f32),
        ),
    )(part1, den1, W2, a2_src.reshape(C, 1), a2_dst.reshape(C, 1))

    part2 = _sc_layer16(t2, es2.reshape(N), ed2.reshape(N), src, dst, z16)

    h2, logp = pl.pallas_call(
        _fin,
        out_shape=(
            jax.ShapeDtypeStruct((N, C), _f32),
            jax.ShapeDtypeStruct((N, C), _f32),
        ),
    )(part2)

    return logp, (h1, h2)


# speedup vs baseline: 1.2073x; 1.1453x over previous
import functools

import jax
import jax.numpy as jnp
from jax import lax
from jax.experimental import pallas as pl
from jax.experimental.pallas import tpu as pltpu
from jax.experimental.pallas import tpu_sc as plsc

N = 10000
E = 320000
D_IN = 128
F1 = 16
C = 7

NC = 2
NS = 16
NW = NC * NS
EPW = E // NW
CH = 400
G = CH // 16
NCHUNK = EPW // CH
NPAD = 10240
RPT = NPAD // NS

_f32 = jnp.float32


def _sc_layer(roww, with_den):
    mesh = plsc.VectorSubcoreMesh(
        core_axis_name="c", subcore_axis_name="s", num_cores=NC, num_subcores=NS
    )

    nbuf = 2

    out_type = [jax.ShapeDtypeStruct((NC, NPAD, roww), _f32)]
    scratch = [
        pltpu.VMEM((N,), _f32),
        pltpu.VMEM((N,), _f32),
        pltpu.VMEM((NCHUNK, CH), jnp.int32),
        pltpu.VMEM((NCHUNK, CH), jnp.int32),
        pltpu.VMEM((nbuf, CH, roww), _f32),
        pltpu.VMEM((nbuf, CH, roww), _f32),
        pltpu.VMEM_SHARED((NPAD, roww), _f32),
        [pltpu.SemaphoreType.DMA] * nbuf,
        [pltpu.SemaphoreType.DMA] * nbuf,
        [pltpu.SemaphoreType.DMA] * 6,
    ]
    if with_den:
        out_type.append(jax.ShapeDtypeStruct((NC, NS, N), _f32))
        scratch.insert(7, pltpu.VMEM((N,), _f32))
    else:
        out_type = out_type[0]

    @functools.partial(
        pl.kernel,
        out_type=out_type,
        mesh=mesh,
        scratch_types=scratch,
        compiler_params=pltpu.CompilerParams(
            use_tc_tiling_on_sc=False, needs_layout_passes=False
        ),
    )
    def layer(t_hbm, esed_hbm, src_hbm, dst_hbm, z_hbm, *rest):
        if with_den:
            (zden_hbm, out_hbm, den_hbm,
             es_v, ed_v, src_v, dst_v, gbuf, sbuf, acc_sh, den_v,
             gsem, ssem, psem) = rest
        else:
            (out_hbm,
             es_v, ed_v, src_v, dst_v, gbuf, sbuf, acc_sh,
             gsem, ssem, psem) = rest
        cid = lax.axis_index("c")
        sid = lax.axis_index("s")
        wid = cid * NS + sid

        setups = [
            (esed_hbm.at[0], es_v),
            (esed_hbm.at[1], ed_v),
            (src_hbm.at[wid], src_v),
            (dst_hbm.at[wid], dst_v),
            (z_hbm, acc_sh.at[pl.ds(sid * RPT, RPT)]),
        ]
        if with_den:
            setups.append((zden_hbm, den_v))
        for k, (s, d) in enumerate(setups):
            pltpu.async_copy(s, d, psem[k])
        for k, (s, d) in enumerate(setups):
            pltpu.make_async_copy(s, d, psem[k]).wait()
        plsc.subcore_barrier()

        def gather(c, b):
            pltpu.async_copy(t_hbm.at[src_v.at[c]], gbuf.at[b], gsem[b])

        offs = list(range(0, roww - 15, 16))
        if roww % 16:
            offs.append(roww - 16)

        def compute(c, b):
            src_row = src_v.at[c]
            dst_row = dst_v.at[c]
            for g in range(G):
                s16 = src_row[pl.ds(g * 16, 16)]
                d16 = dst_row[pl.ds(g * 16, 16)]
                x = plsc.load_gather(es_v, [s16]) + plsc.load_gather(ed_v, [d16])
                ex16 = jnp.exp(jnp.maximum(x, 0.2 * x))
                if with_den:
                    plsc.addupdate_scatter(den_v, [d16], ex16)
                for i in range(16):
                    e = g * 16 + i
                    exi = ex16[i]
                    for h in offs:
                        sbuf[b, e, pl.ds(h, 16)] = (
                            gbuf[b, e, pl.ds(h, 16)] * exi
                        )

        def scatter(c, b):
            pltpu.async_copy(sbuf.at[b], acc_sh.at[dst_v.at[c]], ssem[b], add=True)

        for b in range(nbuf):
            gather(b, b)

        nfull = (NCHUNK - 1) // nbuf

        def step(t, carry):
            for b in range(nbuf):
                c = t * nbuf + b
                pltpu.make_async_copy(t_hbm.at[src_v.at[0]], gbuf.at[b],
                                      gsem[b]).wait()

                @pl.when(t > 0)
                def _():
                    pltpu.make_async_copy(
                        sbuf.at[b], acc_sh.at[dst_v.at[0]], ssem[b]).wait()

                compute(c, b)
                scatter(c, b)

                @pl.when(c + nbuf <= NCHUNK - 1)
                def _():
                    gather(c + nbuf, b)
            return carry

        lax.fori_loop(0, nfull, step, 0)

        last = NCHUNK - 1
        pltpu.make_async_copy(t_hbm.at[src_v.at[0]], gbuf.at[0], gsem[0]).wait()
        pltpu.make_async_copy(sbuf.at[0], acc_sh.at[dst_v.at[0]], ssem[0]).wait()
        compute(last, 0)
        scatter(last, 0)
        for b in range(nbuf):
            pltpu.make_async_copy(sbuf.at[b], acc_sh.at[dst_v.at[0]],
                                  ssem[b]).wait()

        plsc.subcore_barrier()
        outs = [(acc_sh.at[pl.ds(sid * RPT, RPT)],
                 out_hbm.at[cid, pl.ds(sid * RPT, RPT)])]
        if with_den:
            outs.append((den_v, den_hbm.at[cid, sid]))
        for k, (s, d) in enumerate(outs):
            pltpu.async_copy(s, d, psem[k])
        for k, (s, d) in enumerate(outs):
            pltpu.make_async_copy(s, d, psem[k]).wait()

    return layer


_sc_layer1 = _sc_layer(F1, with_den=True)
_sc_layer16 = _sc_layer(16, with_den=False)


def _prep1(h_ref, w1_ref, a_ref, t1_ref, esed_ref):
    hw = jnp.dot(h_ref[...], w1_ref[...], preferred_element_type=_f32)
    t1_ref[...] = hw
    esed_ref[...] = lax.dot_general(
        a_ref[...], hw, (((1,), (1,)), ((), ())),
        preferred_element_type=_f32)


def _mid(p_ref, den_ref, w2_ref, a_ref, h1_ref, t2_ref, esed_ref):
    p = p_ref[...]
    s = (p[0] + p[1])[:N]
    den = jnp.sum(den_ref[...], axis=(0, 1))
    h1 = s / (den[:, None] + 1e-16)
    h1_ref[...] = h1
    hd = jnp.where(h1 > 0, h1, jnp.exp(h1) - 1.0)
    hw2 = jnp.dot(hd, w2_ref[...], preferred_element_type=_f32)
    ones = jnp.ones((N, 1), _f32)
    zeros = jnp.zeros((N, 16 - C - 1), _f32)
    t2_ref[...] = jnp.concatenate([hw2, ones, zeros], axis=1)
    esed_ref[...] = lax.dot_general(
        a_ref[...], hw2, (((1,), (1,)), ((), ())),
        preferred_element_type=_f32)


def _fin(p_ref, h2_ref, lp_ref):
    p = p_ref[...]
    s = (p[0] + p[1])[:N]
    h2 = s[:, :C] / (s[:, C:C + 1] + 1e-16)
    h2_ref[...] = h2
    m = jnp.max(h2, axis=1, keepdims=True)
    z = h2 - m
    lse = jnp.log(jnp.sum(jnp.exp(z), axis=1, keepdims=True))
    lp_ref[...] = z - lse


def kernel(H, edge_index, W1, a1_src, a1_dst, W2, a2_src, a2_dst):
    src = edge_index[0].reshape(NW, NCHUNK, CH)
    dst = edge_index[1].reshape(NW, NCHUNK, CH)

    a1 = jnp.stack([a1_src, a1_dst])
    a2 = jnp.stack([a2_src, a2_dst])

    t1, esed1 = pl.pallas_call(
        _prep1,
        out_shape=(
            jax.ShapeDtypeStruct((N, F1), _f32),
            jax.ShapeDtypeStruct((2, N), _f32),
        ),
    )(H, W1, a1)

    z16 = jnp.zeros((RPT, 16), _f32)
    zden = jnp.zeros((N,), _f32)
    part1, den1 = _sc_layer1(t1, esed1, src, dst, z16, zden)

    h1, t2, esed2 = pl.pallas_call(
        _mid,
        out_shape=(
            jax.ShapeDtypeStruct((N, F1), _f32),
            jax.ShapeDtypeStruct((N, 16), _f32),
            jax.ShapeDtypeStruct((2, N), _f32),
        ),
    )(part1, den1, W2, a2)

    part2 = _sc_layer16(t2, esed2, src, dst, z16)

    h2, logp = pl.pallas_call(
        _fin,
        out_shape=(
            jax.ShapeDtypeStruct((N, C), _f32),
            jax.ShapeDtypeStruct((N, C), _f32),
        ),
    )(part2)

    return logp, (h1, h2)


# speedup vs baseline: 1.2084x; 1.0009x over previous
import functools

import jax
import jax.numpy as jnp
from jax import lax
from jax.experimental import pallas as pl
from jax.experimental.pallas import tpu as pltpu
from jax.experimental.pallas import tpu_sc as plsc

N = 10000
E = 320000
D_IN = 128
F1 = 16
C = 7

NC = 2
NS = 16
NW = NC * NS
EPW = E // NW
CH = 400
G = CH // 16
NCHUNK = EPW // CH
NPAD = 10240
RPT = NPAD // NS

_f32 = jnp.float32


def _sc_layer(roww, with_den):
    mesh = plsc.VectorSubcoreMesh(
        core_axis_name="c", subcore_axis_name="s", num_cores=NC, num_subcores=NS
    )

    nbuf = 2

    out_type = [jax.ShapeDtypeStruct((NC, NPAD, roww), _f32)]
    scratch = [
        pltpu.VMEM((N,), _f32),
        pltpu.VMEM((N,), _f32),
        pltpu.VMEM((NCHUNK, CH), jnp.int32),
        pltpu.VMEM((NCHUNK, CH), jnp.int32),
        pltpu.VMEM((nbuf, CH, roww), _f32),
        pltpu.VMEM((nbuf, CH, roww), _f32),
        pltpu.VMEM_SHARED((NPAD, roww), _f32),
        [pltpu.SemaphoreType.DMA] * nbuf,
        [pltpu.SemaphoreType.DMA] * nbuf,
        [pltpu.SemaphoreType.DMA] * 6,
    ]
    if with_den:
        out_type.append(jax.ShapeDtypeStruct((NC, NS, NPAD), _f32))
        scratch.insert(7, pltpu.VMEM((N,), _f32))
    else:
        out_type = out_type[0]

    @functools.partial(
        pl.kernel,
        out_type=out_type,
        mesh=mesh,
        scratch_types=scratch,
        compiler_params=pltpu.CompilerParams(
            use_tc_tiling_on_sc=False, needs_layout_passes=False
        ),
    )
    def layer(t_hbm, esed_hbm, src_hbm, dst_hbm, z_hbm, *rest):
        if with_den:
            (zden_hbm, out_hbm, den_hbm,
             es_v, ed_v, src_v, dst_v, gbuf, sbuf, acc_sh, den_v,
             gsem, ssem, psem) = rest
        else:
            (out_hbm,
             es_v, ed_v, src_v, dst_v, gbuf, sbuf, acc_sh,
             gsem, ssem, psem) = rest
        cid = lax.axis_index("c")
        sid = lax.axis_index("s")
        wid = cid * NS + sid

        setups = [
            (esed_hbm.at[0], es_v),
            (esed_hbm.at[1], ed_v),
            (src_hbm.at[wid], src_v),
            (dst_hbm.at[wid], dst_v),
            (z_hbm, acc_sh.at[pl.ds(sid * RPT, RPT)]),
        ]
        if with_den:
            setups.append((zden_hbm, den_v))
        for k, (s, d) in enumerate(setups):
            pltpu.async_copy(s, d, psem[k])
        for k, (s, d) in enumerate(setups):
            pltpu.make_async_copy(s, d, psem[k]).wait()
        plsc.subcore_barrier()

        def gather(c, b):
            pltpu.async_copy(t_hbm.at[src_v.at[c]], gbuf.at[b], gsem[b])

        offs = list(range(0, roww - 15, 16))
        if roww % 16:
            offs.append(roww - 16)

        def compute(c, b):
            src_row = src_v.at[c]
            dst_row = dst_v.at[c]
            for g in range(G):
                s16 = src_row[pl.ds(g * 16, 16)]
                d16 = dst_row[pl.ds(g * 16, 16)]
                x = plsc.load_gather(es_v, [s16]) + plsc.load_gather(ed_v, [d16])
                ex16 = jnp.exp(jnp.maximum(x, 0.2 * x))
                if with_den:
                    plsc.addupdate_scatter(den_v, [d16], ex16)
                for i in range(16):
                    e = g * 16 + i
                    exi = ex16[i]
                    for h in offs:
                        sbuf[b, e, pl.ds(h, 16)] = (
                            gbuf[b, e, pl.ds(h, 16)] * exi
                        )

        def scatter(c, b):
            pltpu.async_copy(sbuf.at[b], acc_sh.at[dst_v.at[c]], ssem[b], add=True)

        for b in range(nbuf):
            gather(b, b)

        nfull = (NCHUNK - 1) // nbuf

        def step(t, carry):
            for b in range(nbuf):
                c = t * nbuf + b
                pltpu.make_async_copy(t_hbm.at[src_v.at[0]], gbuf.at[b],
                                      gsem[b]).wait()

                @pl.when(t > 0)
                def _():
                    pltpu.make_async_copy(
                        sbuf.at[b], acc_sh.at[dst_v.at[0]], ssem[b]).wait()

                compute(c, b)
                scatter(c, b)

                @pl.when(c + nbuf <= NCHUNK - 1)
                def _():
                    gather(c + nbuf, b)
            return carry

        lax.fori_loop(0, nfull, step, 0)

        last = NCHUNK - 1
        pltpu.make_async_copy(t_hbm.at[src_v.at[0]], gbuf.at[0], gsem[0]).wait()
        pltpu.make_async_copy(sbuf.at[0], acc_sh.at[dst_v.at[0]], ssem[0]).wait()
        compute(last, 0)
        scatter(last, 0)
        for b in range(nbuf):
            pltpu.make_async_copy(sbuf.at[b], acc_sh.at[dst_v.at[0]],
                                  ssem[b]).wait()

        plsc.subcore_barrier()
        outs = [(acc_sh.at[pl.ds(sid * RPT, RPT)],
                 out_hbm.at[cid, pl.ds(sid * RPT, RPT)])]
        if with_den:
            outs.append((den_v, den_hbm.at[cid, sid, pl.ds(0, N)]))
        for k, (s, d) in enumerate(outs):
            pltpu.async_copy(s, d, psem[k])
        for k, (s, d) in enumerate(outs):
            pltpu.make_async_copy(s, d, psem[k]).wait()

    return layer


_sc_layer1 = _sc_layer(F1, with_den=True)
_sc_layer16 = _sc_layer(16, with_den=False)


def _prep1(h_ref, w1_ref, a_ref, t1_ref, esed_ref):
    hw = jnp.dot(h_ref[...], w1_ref[...], preferred_element_type=_f32)
    t1_ref[...] = hw
    esed_ref[...] = lax.dot_general(
        a_ref[...], hw, (((1,), (1,)), ((), ())),
        preferred_element_type=_f32)


def _mid(p_ref, den_ref, w2_ref, a_ref, h1_ref, t2_ref, esed_ref):
    p = p_ref[...]
    s = (p[0] + p[1])[:N]
    den = jnp.sum(den_ref[...], axis=(0, 1))[:N]
    h1 = s / (den[:, None] + 1e-16)
    h1_ref[...] = h1
    hd = jnp.where(h1 > 0, h1, jnp.exp(h1) - 1.0)
    hw2 = jnp.dot(hd, w2_ref[...], preferred_element_type=_f32)
    ones = jnp.ones((N, 1), _f32)
    zeros = jnp.zeros((N, 16 - C - 1), _f32)
    t2_ref[...] = jnp.concatenate([hw2, ones, zeros], axis=1)
    esed_ref[...] = lax.dot_general(
        a_ref[...], hw2, (((1,), (1,)), ((), ())),
        preferred_element_type=_f32)


def _fin(p_ref, h2_ref, lp_ref):
    p = p_ref[...]
    s = (p[0] + p[1])[:N]
    h2 = s[:, :C] / (s[:, C:C + 1] + 1e-16)
    h2_ref[...] = h2
    m = jnp.max(h2, axis=1, keepdims=True)
    z = h2 - m
    lse = jnp.log(jnp.sum(jnp.exp(z), axis=1, keepdims=True))
    lp_ref[...] = z - lse


def kernel(H, edge_index, W1, a1_src, a1_dst, W2, a2_src, a2_dst):
    src = edge_index[0].reshape(NW, NCHUNK, CH)
    dst = edge_index[1].reshape(NW, NCHUNK, CH)

    a1 = jnp.stack([a1_src, a1_dst])
    a2 = jnp.stack([a2_src, a2_dst])

    t1, esed1 = pl.pallas_call(
        _prep1,
        out_shape=(
            jax.ShapeDtypeStruct((N, F1), _f32),
            jax.ShapeDtypeStruct((2, N), _f32),
        ),
    )(H, W1, a1)

    z16 = jnp.zeros((RPT, 16), _f32)
    zden = jnp.zeros((N,), _f32)
    part1, den1 = _sc_layer1(t1, esed1, src, dst, z16, zden)

    h1, t2, esed2 = pl.pallas_call(
        _mid,
        out_shape=(
            jax.ShapeDtypeStruct((N, F1), _f32),
            jax.ShapeDtypeStruct((N, 16), _f32),
            jax.ShapeDtypeStruct((2, N), _f32),
        ),
    )(part1, den1, W2, a2)

    part2 = _sc_layer16(t2, esed2, src, dst, z16)

    h2, logp = pl.pallas_call(
        _fin,
        out_shape=(
            jax.ShapeDtypeStruct((N, C), _f32),
            jax.ShapeDtypeStruct((N, C), _f32),
        ),
    )(part2)

    return logp, (h1, h2)
